# Initial kernel scaffold; baseline (speedup 1.0000x reference)
import jax
import jax.numpy as jnp
from jax.experimental import pallas as pl


def kernel(inputs, text_num, W, b, gamma, beta):
    raise NotImplementedError("write your pallas kernel here")


# speedup vs baseline: 1.9662x; 1.9662x over previous
import jax
import jax.numpy as jnp
from jax.experimental import pallas as pl
from jax.experimental.pallas import tpu as pltpu

B, S, D_IN, D_MODEL = 16, 2048, 1024, 1024
BS = 256


def _body(tn_ref, x_ref, w_ref, b_ref, g_ref, be_ref, o_ref):
    bi = pl.program_id(0)
    si = pl.program_id(1)
    tn = tn_ref[bi]
    start = si * BS

    @pl.when(start < tn)
    def _compute():
        x = x_ref[0].astype(jnp.bfloat16)
        h = jnp.dot(x, w_ref[...], preferred_element_type=jnp.float32)
        h = h + b_ref[...]
        mu = jnp.mean(h, axis=-1, keepdims=True)
        var = jnp.mean((h - mu) ** 2, axis=-1, keepdims=True)
        hn = (h - mu) * jax.lax.rsqrt(var + 1e-5) * g_ref[...] + be_ref[...]
        r = jnp.maximum(hn, 0.0)
        idx = start + jax.lax.broadcasted_iota(jnp.int32, (BS, 1), 0)
        o_ref[0] = jnp.where(idx < tn, r, 0.0)

    @pl.when(start >= tn)
    def _zero():
        o_ref[0] = jnp.zeros((BS, D_MODEL), jnp.float32)


def kernel(inputs, text_num, W, b, gamma, beta):
    w_bf16 = W.astype(jnp.bfloat16)
    b2 = b.reshape(1, D_MODEL)
    g2 = gamma.reshape(1, D_MODEL)
    be2 = beta.reshape(1, D_MODEL)

    grid_spec = pltpu.PrefetchScalarGridSpec(
        num_scalar_prefetch=1,
        grid=(B, S // BS),
        in_specs=[
            pl.BlockSpec((1, BS, D_IN), lambda bi, si, tn: (bi, si, 0)),
            pl.BlockSpec((D_IN, D_MODEL), lambda bi, si, tn: (0, 0)),
            pl.BlockSpec((1, D_MODEL), lambda bi, si, tn: (0, 0)),
            pl.BlockSpec((1, D_MODEL), lambda bi, si, tn: (0, 0)),
            pl.BlockSpec((1, D_MODEL), lambda bi, si, tn: (0, 0)),
        ],
        out_specs=pl.BlockSpec((1, BS, D_MODEL), lambda bi, si, tn: (bi, si, 0)),
    )
    return pl.pallas_call(
        _body,
        grid_spec=grid_spec,
        out_shape=jax.ShapeDtypeStruct((B, S, D_MODEL), jnp.float32),
        compiler_params=pltpu.CompilerParams(
            dimension_semantics=("parallel", "arbitrary"),
        ),
    )(text_num, inputs, w_bf16, b2, g2, be2)
